# baseline (speedup 1.0000x reference)
import functools
import math

import jax
import jax.numpy as jnp
from jax import lax
from jax.experimental import pallas as pl
from jax.experimental.pallas import tpu as pltpu
from jax.experimental.pallas import tpu_sc as plsc

NC = 2
NS = 16
NW = NC * NS
CHUNK = 128
E_PAD = 327680
BN = 256
HEADS = 4
RATIO = 0.5
LVL = 3


def _ceil_to(x, m):
    return ((x + m - 1) // m) * m


def _acc_rows(n):
    return _ceil_to(n + 1, 256)


def _pad_rows(x, n_acc):
    return jnp.pad(x, ((0, n_acc - x.shape[0]), (0, 0)))


@functools.lru_cache(maxsize=None)
def _make_degree(n_acc, e_pad):
    epw = e_pad // NW
    nblk = epw // 2048
    mesh = plsc.VectorSubcoreMesh(core_axis_name="c", subcore_axis_name="s")

    @functools.partial(
        pl.kernel,
        out_type=jax.ShapeDtypeStruct((NW, n_acc), jnp.float32),
        mesh=mesh,
        scratch_types=[
            pltpu.VMEM((2048,), jnp.int32),
            pltpu.VMEM((n_acc,), jnp.float32),
        ],
        compiler_params=pltpu.CompilerParams(needs_layout_passes=False),
    )
    def deg_kernel(cols_hbm, degp_hbm, colbuf, dloc):
        c = lax.axis_index("c")
        s = lax.axis_index("s")
        wid = c * NS + s

        def zero(i, _):
            dloc[pl.ds(i * 16, 16)] = jnp.zeros((16,), jnp.float32)
            return 0
        lax.fori_loop(0, n_acc // 16, zero, 0)

        ones = jnp.full((16,), 1.0, jnp.float32)

        def blk(bi, _):
            pltpu.sync_copy(cols_hbm.at[pl.ds(wid * epw + bi * 2048, 2048)], colbuf)

            def inner(j, _):
                cv = colbuf[pl.ds(j * 16, 16)]
                plsc.addupdate_scatter(dloc, [cv], ones)
                return 0
            lax.fori_loop(0, 2048 // 16, inner, 0)
            return 0
        lax.fori_loop(0, nblk, blk, 0)

        pltpu.sync_copy(dloc, degp_hbm.at[wid])

    return deg_kernel


@functools.lru_cache(maxsize=None)
def _make_propagate(n_acc, e_pad):
    epw = e_pad // NW
    n_chunks = epw // CHUNK
    rpt = n_acc // NS
    mesh = plsc.VectorSubcoreMesh(core_axis_name="c", subcore_axis_name="s")

    @functools.partial(
        pl.kernel,
        out_type=jax.ShapeDtypeStruct((NC, n_acc, 128), jnp.float32),
        mesh=mesh,
        scratch_types=[
            pltpu.VMEM((CHUNK,), jnp.int32),
            pltpu.VMEM((CHUNK,), jnp.int32),
            pltpu.VMEM((CHUNK, 128), jnp.float32),
            pltpu.VMEM((16, 128), jnp.float32),
            pltpu.VMEM_SHARED((n_acc, 128), jnp.float32),
            pltpu.SemaphoreType.DMA,
        ],
        compiler_params=pltpu.CompilerParams(needs_layout_passes=False),
    )
    def prop_kernel(h_hbm, rowg_hbm, cols_hbm, out_hbm, idxg, idxs, rows, zbuf,
                    acc, sem):
        c = lax.axis_index("c")
        s = lax.axis_index("s")
        wid = c * NS + s

        def zrow(r, _):
            def zcol(j, _):
                zbuf[r, pl.ds(j * 16, 16)] = jnp.zeros((16,), jnp.float32)
                return 0
            return lax.fori_loop(0, 8, zcol, 0)
        lax.fori_loop(0, 16, zrow, 0)

        def zacc(k, _):
            pltpu.sync_copy(zbuf, acc.at[pl.ds(s * rpt + k * 16, 16)])
            return 0
        lax.fori_loop(0, rpt // 16, zacc, 0)
        plsc.subcore_barrier()

        base0 = wid * epw

        def body(i, _):
            base = base0 + i * CHUNK
            pltpu.sync_copy(rowg_hbm.at[pl.ds(base, CHUNK)], idxg)
            pltpu.sync_copy(cols_hbm.at[pl.ds(base, CHUNK)], idxs)
            pltpu.async_copy(h_hbm.at[idxg], rows, sem).wait()
            pltpu.sync_copy(rows, acc.at[idxs], add=True)
            return 0
        lax.fori_loop(0, n_chunks, body, 0)
        plsc.subcore_barrier()

        pltpu.sync_copy(acc.at[pl.ds(s * rpt, rpt)],
                        out_hbm.at[c, pl.ds(s * rpt, rpt)])

    return prop_kernel


def _mm_scale(x_pad, w, degp, fill):
    n_acc = x_pad.shape[0]

    def body(x_ref, w_ref, d_ref, o_ref):
        deg = jnp.sum(d_ref[...], axis=0) + fill
        dinv = lax.rsqrt(deg)
        h = jnp.dot(x_ref[...], w_ref[...], preferred_element_type=jnp.float32)
        o_ref[...] = h * dinv[:, None]

    return pl.pallas_call(
        body,
        grid=(n_acc // BN,),
        in_specs=[pl.BlockSpec((BN, 128), lambda i: (i, 0)),
                  pl.BlockSpec((128, 128), lambda i: (0, 0)),
                  pl.BlockSpec((NW, BN), lambda i: (0, i))],
        out_specs=pl.BlockSpec((BN, 128), lambda i: (i, 0)),
        out_shape=jax.ShapeDtypeStruct((n_acc, 128), jnp.float32),
    )(x_pad, w, degp)


def _finish(accp, hp, degp, b, fill, relu):
    n_acc = hp.shape[0]

    def body(a_ref, h_ref, d_ref, b_ref, o_ref):
        deg = jnp.sum(d_ref[...], axis=0) + fill
        dinv = lax.rsqrt(deg)
        o = (a_ref[0] + a_ref[1] + fill * h_ref[...]) * dinv[:, None] + b_ref[...]
        if relu:
            o = jnp.maximum(o, 0.0)
        o_ref[...] = o

    return pl.pallas_call(
        body,
        grid=(n_acc // BN,),
        in_specs=[pl.BlockSpec((2, BN, 128), lambda i: (0, i, 0)),
                  pl.BlockSpec((BN, 128), lambda i: (i, 0)),
                  pl.BlockSpec((NW, BN), lambda i: (0, i)),
                  pl.BlockSpec((1, 128), lambda i: (0, 0))],
        out_specs=pl.BlockSpec((BN, 128), lambda i: (i, 0)),
        out_shape=jax.ShapeDtypeStruct((n_acc, 128), jnp.float32),
    )(accp, hp, degp, b.reshape(1, 128))


def _prep_edges(ei, ew, n):
    e = ei.shape[1]
    valid = ew > 0
    rowg = jnp.where(valid, ei[0], 0).astype(jnp.int32)
    cols = jnp.where(valid, ei[1], n).astype(jnp.int32)
    pad = E_PAD - e
    rowg = jnp.concatenate([rowg, jnp.zeros((pad,), jnp.int32)])
    cols = jnp.concatenate([cols, jnp.full((pad,), n, jnp.int32)])
    return rowg, cols


def _gcn_sc(x_pad, rowg, cols, degp, p, fill, relu):
    n_acc = x_pad.shape[0]
    hp = _mm_scale(x_pad, p['W'], degp, fill)
    accp = _make_propagate(n_acc, E_PAD)(hp, rowg, cols)
    return _finish(accp, hp, degp, p['b'], fill, relu)


def _attn_tail(Qp, Kd, Vd, p):
    dv = Qp.shape[-1]
    split = lambda t: jnp.concatenate(jnp.split(t, HEADS, axis=2), axis=0)
    Q_, K_, V_ = split(Qp), split(Kd), split(Vd)
    A = jax.nn.softmax(jnp.matmul(Q_, jnp.swapaxes(K_, 1, 2)) / math.sqrt(dv),
                       axis=-1)
    out = Q_ + jnp.matmul(A, V_)
    out = jnp.concatenate(jnp.split(out, HEADS, axis=0), axis=2)
    return out + jax.nn.relu(out @ p['o']['W'] + p['o']['b'])


def _mab_dense(Q, K, p):
    Qp = Q @ p['q']['W'] + p['q']['b']
    Kd = K @ p['k']['W'] + p['k']['b']
    Vd = K @ p['v']['W'] + p['v']['b']
    return _attn_tail(Qp, Kd, Vd, p)


def kernel(x, edge_index, edge_weight, params):
    n0 = x.shape[0]
    ew = jnp.ones((edge_index.shape[1],), x.dtype)
    na0 = _acc_rows(n0)
    rowg0, cols0 = _prep_edges(edge_index, ew, n0)
    degp0 = _make_degree(na0, E_PAD)(cols0)

    xp = _pad_rows(x, na0)
    xp = _gcn_sc(xp, rowg0, cols0, degp0, params['down'][0], 2.0, True)

    xs = [xp]
    ns = [n0]
    rcs = [(rowg0, cols0)]
    degps = [degp0]
    perms = []

    cur_ei, cur_ew, n_cur = edge_index, ew, n0
    for i in range(1, LVL + 1):
        xf = xp[:n_cur]
        w = params['pool'][i - 1]
        score = jnp.tanh((xf @ w) / jnp.linalg.norm(w))
        k = int(math.ceil(RATIO * n_cur))
        vals, perm = lax.top_k(score, k)
        x_new = xf[perm] * vals[:, None]
        node_idx = jnp.full((n_cur,), -1, jnp.int32).at[perm].set(
            jnp.arange(k, dtype=jnp.int32))
        nr = node_idx[cur_ei[0]]
        ncol = node_idx[cur_ei[1]]
        valid = (nr >= 0) & (ncol >= 0)
        cur_ei = jnp.stack([jnp.where(valid, nr, 0),
                            jnp.where(valid, ncol, 0)]).astype(cur_ei.dtype)
        cur_ew = jnp.where(valid, cur_ew, 0.0)
        n_cur = k

        na = _acc_rows(k)
        rowg, cols = _prep_edges(cur_ei, cur_ew, k)
        degp = _make_degree(na, E_PAD)(cols)
        xp = _pad_rows(x_new, na)
        xp = _gcn_sc(xp, rowg, cols, degp, params['down'][i], 2.0, True)
        if i < LVL:
            xs.append(xp)
            ns.append(k)
            rcs.append((rowg, cols))
            degps.append(degp)
        perms.append(perm)

    for i in range(LVL):
        j = LVL - 1 - i
        kj = perms[j].shape[0]
        xt = xp[:kj]
        up = jnp.zeros((ns[j], 128), jnp.float32).at[perms[j]].set(xt)
        xsum = xs[j][:ns[j]] + up
        xp = _pad_rows(xsum, _acc_rows(ns[j]))
        rowg, cols = rcs[j]
        xp = _gcn_sc(xp, rowg, cols, degps[j], params['up'][i], 2.0, i < LVL - 1)

    g = params['gmt']
    xt = xp[:n0]
    h = xt @ g['lin1']['W'] + g['lin1']['b']
    hp_pad = _pad_rows(h, na0)
    Kd = _gcn_sc(hp_pad, rowg0, cols0, degp0, g['mab_g']['k'], 1.0, False)[:n0][None]
    Vd = _gcn_sc(hp_pad, rowg0, cols0, degp0, g['mab_g']['v'], 1.0, False)[:n0][None]
    Qp = g['S_g'] @ g['mab_g']['q']['W'] + g['mab_g']['q']['b']
    bx = _attn_tail(Qp, Kd, Vd, g['mab_g'])
    bx = _mab_dense(bx, bx, g['mab_s'])
    bx = _mab_dense(g['S_i'], bx, g['mab_i'])
    out = bx[:, 0, :] @ g['lin2']['W'] + g['lin2']['b']
    return out @ params['final']['W'] + params['final']['b']
